# baseline (speedup 1.0000x reference)
import jax
import jax.numpy as jnp
from jax import lax
from jax.experimental import pallas as pl
from jax.experimental.pallas import tpu as pltpu
from jax.experimental.pallas import tpu_sc as plsc

D = 32
K = 8
ND = D * K
W = 2 * ND + K
SW = ND + K
N = 16384

NC = 2
NS = 16
NW = NC * NS
RPW = N // NW
CH = 64
NCH = RPW // CH


def _sc_body(x_hbm, mean_hbm, std_hbm, pi_hbm,
             s0, s1, o0, o1, p0, p1,
             in0, in1, so0, so1, qo0, qo1, sem_mean):
    wid = lax.axis_index("s") * NC + lax.axis_index("c")
    base = wid * RPW

    mean_cp = pltpu.make_async_copy(
        x_hbm.at[pl.ds(base, RPW), pl.ds(0, ND)],
        mean_hbm.at[pl.ds(base, RPW)], sem_mean)
    mean_cp.start()

    S, O, P = [s0, s1], [o0, o1], [p0, p1]
    IN, SO, QO = [in0, in1], [so0, so1], [qo0, qo1]
    lane = lax.iota(jnp.int32, 16)
    hin, hso, hqo = {}, {}, {}

    def start_in(c):
        b = c & 1
        h = pltpu.make_async_copy(
            x_hbm.at[pl.ds(base + c * CH, CH), pl.ds(ND, SW)], S[b], IN[b])
        h.start()
        hin[c] = h

    start_in(0)
    start_in(1)

    for c in range(NCH):
        b = c & 1
        hin[c].wait()
        if c >= 2:
            hso[c - 2].wait()
            hqo[c - 2].wait()

        def row(r, carry, _b=b):
            for j in range(ND // 16):
                v = S[_b][r, pl.ds(j * 16, 16)]
                O[_b][r, pl.ds(j * 16, 16)] = 1.0 / (1.0 + jnp.exp(-v))
            return carry
        lax.fori_loop(0, CH, row, 0)

        for g in range(CH // 16):
            rows = g * 16 + lane
            cols = [plsc.load_gather(S[b], [rows, jnp.full((16,), ND + k, jnp.int32)])
                    for k in range(K)]
            m = cols[0]
            for k in range(1, K):
                m = jnp.maximum(m, cols[k])
            es = [jnp.exp(v - m) for v in cols]
            ssum = es[0]
            for k in range(1, K):
                ssum = ssum + es[k]
            inv = 1.0 / ssum
            for k in range(K):
                plsc.store_scatter(P[b], [rows, jnp.full((16,), k, jnp.int32)],
                                   es[k] * inv)

        h = pltpu.make_async_copy(O[b], std_hbm.at[pl.ds(base + c * CH, CH)], SO[b])
        h.start()
        hso[c] = h
        h = pltpu.make_async_copy(P[b], pi_hbm.at[pl.ds(base + c * CH, CH)], QO[b])
        h.start()
        hqo[c] = h
        if c + 2 < NCH:
            start_in(c + 2)

    for c in (NCH - 2, NCH - 1):
        hso[c].wait()
        hqo[c].wait()
    mean_cp.wait()


def kernel(x):
    mean2d, std2d, pi = pl.kernel(
        _sc_body,
        mesh=plsc.VectorSubcoreMesh(core_axis_name="c", subcore_axis_name="s"),
        out_type=[
            jax.ShapeDtypeStruct((N, ND), jnp.float32),
            jax.ShapeDtypeStruct((N, ND), jnp.float32),
            jax.ShapeDtypeStruct((N, K), jnp.float32),
        ],
        scratch_types=[
            pltpu.VMEM((CH, SW), jnp.float32),
            pltpu.VMEM((CH, SW), jnp.float32),
            pltpu.VMEM((CH, ND), jnp.float32),
            pltpu.VMEM((CH, ND), jnp.float32),
            pltpu.VMEM((CH, K), jnp.float32),
            pltpu.VMEM((CH, K), jnp.float32),
            pltpu.SemaphoreType.DMA,
            pltpu.SemaphoreType.DMA,
            pltpu.SemaphoreType.DMA,
            pltpu.SemaphoreType.DMA,
            pltpu.SemaphoreType.DMA,
            pltpu.SemaphoreType.DMA,
            pltpu.SemaphoreType.DMA,
        ],
        compiler_params=pltpu.CompilerParams(
            needs_layout_passes=False, use_tc_tiling_on_sc=False),
    )(x)
    return (mean2d.reshape(N, D, K), std2d.reshape(N, D, K), pi)


# speedup vs baseline: 1.3315x; 1.3315x over previous
import jax
import jax.numpy as jnp
from jax import lax
from jax.experimental import pallas as pl
from jax.experimental.pallas import tpu as pltpu
from jax.experimental.pallas import tpu_sc as plsc

D = 32
K = 8
ND = D * K
W = 2 * ND + K
N = 16384

NC = 2
NS = 16
NW = NC * NS
RPW = N // NW

BM = 2048


def _sc_mean(x_hbm, mean_hbm, sem):
    wid = lax.axis_index("s") * NC + lax.axis_index("c")
    base = wid * RPW
    h = pltpu.make_async_copy(
        x_hbm.at[pl.ds(base, RPW), pl.ds(0, ND)],
        mean_hbm.at[pl.ds(base, RPW)], sem)
    h.start()
    h.wait()


def _tc_body(xs_ref, xp_ref, std_ref, pi_ref):
    std_ref[...] = jax.nn.sigmoid(xs_ref[...])
    xp = xp_ref[...]
    col = lax.broadcasted_iota(jnp.int32, xp.shape, 1)
    logits = jnp.where(col < K, xp, -jnp.inf)
    m = jnp.max(logits, axis=-1, keepdims=True)
    e = jnp.exp(logits - m)
    s = jnp.sum(e, axis=-1, keepdims=True)
    pi_ref[...] = (e / s)[:, :K]


def kernel(x):
    mean2d = pl.kernel(
        _sc_mean,
        mesh=plsc.VectorSubcoreMesh(core_axis_name="c", subcore_axis_name="s"),
        out_type=jax.ShapeDtypeStruct((N, ND), jnp.float32),
        scratch_types=[pltpu.SemaphoreType.DMA],
        compiler_params=pltpu.CompilerParams(
            needs_layout_passes=False, use_tc_tiling_on_sc=True),
    )(x)

    std2d, pi = pl.pallas_call(
        _tc_body,
        grid=(N // BM,),
        in_specs=[
            pl.BlockSpec((BM, ND), lambda i: (i, 1)),
            pl.BlockSpec((BM, 128), lambda i: (i, 4)),
        ],
        out_specs=[
            pl.BlockSpec((BM, ND), lambda i: (i, 0)),
            pl.BlockSpec((BM, K), lambda i: (i, 0)),
        ],
        out_shape=[
            jax.ShapeDtypeStruct((N, ND), jnp.float32),
            jax.ShapeDtypeStruct((N, K), jnp.float32),
        ],
    )(x, x)

    return (mean2d.reshape(N, D, K), std2d.reshape(N, D, K), pi)


# speedup vs baseline: 7.3666x; 5.5327x over previous
import jax
import jax.numpy as jnp
from jax import lax
from jax.experimental import pallas as pl

D = 32
K = 8
ND = D * K
W = 2 * ND + K
N = 16384

BM = 2048


def _tc_body(xm_ref, xs_ref, xp_ref, mean_ref, std_ref, pi_ref):
    mean_ref[...] = xm_ref[...]
    std_ref[...] = jax.nn.sigmoid(xs_ref[...])
    xp = xp_ref[...]
    col = lax.broadcasted_iota(jnp.int32, xp.shape, 1)
    logits = jnp.where(col < K, xp, -jnp.inf)
    m = jnp.max(logits, axis=-1, keepdims=True)
    e = jnp.exp(logits - m)
    s = jnp.sum(e, axis=-1, keepdims=True)
    pi_ref[...] = (e / s)[:, :K]


def kernel(x):
    mean2d, std2d, pi = pl.pallas_call(
        _tc_body,
        grid=(N // BM,),
        in_specs=[
            pl.BlockSpec((BM, ND), lambda i: (i, 0)),
            pl.BlockSpec((BM, ND), lambda i: (i, 1)),
            pl.BlockSpec((BM, 128), lambda i: (i, 4)),
        ],
        out_specs=[
            pl.BlockSpec((BM, ND), lambda i: (i, 0)),
            pl.BlockSpec((BM, ND), lambda i: (i, 0)),
            pl.BlockSpec((BM, K), lambda i: (i, 0)),
        ],
        out_shape=[
            jax.ShapeDtypeStruct((N, ND), jnp.float32),
            jax.ShapeDtypeStruct((N, ND), jnp.float32),
            jax.ShapeDtypeStruct((N, K), jnp.float32),
        ],
    )(x, x, x)
    return (mean2d.reshape(N, D, K), std2d.reshape(N, D, K), pi)


# speedup vs baseline: 31.6957x; 4.3026x over previous
import jax
import jax.numpy as jnp
from jax import lax
from jax.experimental import pallas as pl

D = 32
K = 8
ND = D * K
W = 2 * ND + K
N = 16384

BN = 2048


def _tc_body(xm_ref, xs_ref, xp_ref, mean_ref, std_ref, pi_ref):
    mean_ref[...] = xm_ref[...]
    std_ref[...] = jax.nn.sigmoid(xs_ref[...])
    logits = xp_ref[...]
    m = jnp.max(logits, axis=0, keepdims=True)
    e = jnp.exp(logits - m)
    s = jnp.sum(e, axis=0, keepdims=True)
    pi_ref[...] = e / s


def kernel(x):
    xt = jnp.transpose(x)
    mean_t, std_t, pi_t = pl.pallas_call(
        _tc_body,
        grid=(N // BN,),
        in_specs=[
            pl.BlockSpec((ND, BN), lambda j: (0, j)),
            pl.BlockSpec((ND, BN), lambda j: (1, j)),
            pl.BlockSpec((K, BN), lambda j: (2 * ND // K, j)),
        ],
        out_specs=[
            pl.BlockSpec((ND, BN), lambda j: (0, j)),
            pl.BlockSpec((ND, BN), lambda j: (0, j)),
            pl.BlockSpec((K, BN), lambda j: (0, j)),
        ],
        out_shape=[
            jax.ShapeDtypeStruct((ND, N), jnp.float32),
            jax.ShapeDtypeStruct((ND, N), jnp.float32),
            jax.ShapeDtypeStruct((K, N), jnp.float32),
        ],
    )(xt, xt, xt)
    mean = jnp.transpose(mean_t).reshape(N, D, K)
    std = jnp.transpose(std_t).reshape(N, D, K)
    pi = jnp.transpose(pi_t)
    return (mean, std, pi)
